# Initial kernel scaffold; baseline (speedup 1.0000x reference)
import jax
import jax.numpy as jnp
from jax.experimental import pallas as pl


def kernel(edges, negs, neigh_table, features, W1, W_asp):
    raise NotImplementedError("write your pallas kernel here")


# speedup vs baseline: 2.2282x; 2.2282x over previous
import functools

import jax
import jax.numpy as jnp
from jax import lax
from jax.experimental import pallas as pl
from jax.experimental.pallas import tpu as pltpu
from jax.experimental.pallas import tpu_sc as plsc

N_NODES = 10000
D_FEAT = 256
S_NEIGH = 16
D_HIDDEN = 256
K_ASP = 8
D_OUT = 128
NUM_NEGS = 20
BATCH = 256
ALPHA = 1.0
BETA = 1.0
GAMMA = 0.1

M_PAD = 24
ROWS = BATCH * M_PAD
NUM_CORES = 2
NUM_SUBCORES = 16
NW = NUM_CORES * NUM_SUBCORES
RPW = ROWS // NW
CH = 16
NCH = RPW // CH
LANES = 16

BB = 32
TC_ROWS = BB * M_PAD
GRID = BATCH // BB


def _sc_gather_body(ids_hbm, ntp_hbm, feat_hbm, out_hbm,
                    idx_v, nidc_v, cidx_v, fidx_v, selfb_v, nrows_v, outc_v,
                    sem_s, sem_a, sem_b):
    wid = lax.axis_index("s") * NUM_CORES + lax.axis_index("c")
    base = wid * RPW

    pltpu.sync_copy(ids_hbm.at[pl.ds(base, RPW)], idx_v)

    def chunk(c, carry):
        cb = c * CH
        cidx_v[:] = idx_v[pl.ds(cb, CH)]
        pltpu.async_copy(ntp_hbm.at[cidx_v], nidc_v, sem_s).wait()
        for j in range(CH):
            fidx_v[j // 8, pl.ds((j % 8) * LANES, LANES)] = nidc_v[j, :LANES]
        cp_s = pltpu.async_copy(feat_hbm.at[cidx_v], selfb_v, sem_s)
        cp_a = pltpu.async_copy(feat_hbm.at[fidx_v.at[0]],
                                nrows_v.at[pl.ds(0, 128)], sem_a)
        cp_b = pltpu.async_copy(feat_hbm.at[fidx_v.at[1]],
                                nrows_v.at[pl.ds(128, 128)], sem_b)
        cp_s.wait()
        cp_a.wait()
        cp_b.wait()

        def slot(j, carry2):
            def col(g, carry3):
                sl = pl.ds(g * LANES, LANES)
                acc = nrows_v[j * S_NEIGH, sl]
                for s in range(1, S_NEIGH):
                    acc = acc + nrows_v[j * S_NEIGH + s, sl]
                outc_v[j, pl.ds(D_FEAT + g * LANES, LANES)] = acc
                outc_v[j, sl] = selfb_v[j, sl]
                return carry3
            return lax.fori_loop(0, D_FEAT // LANES, col, carry2)
        lax.fori_loop(0, CH, slot, 0)

        pltpu.sync_copy(outc_v, out_hbm.at[pl.ds(base + cb, CH), :])
        return carry
    lax.fori_loop(0, NCH, chunk, 0)


@functools.partial(jax.jit, static_argnums=())
def _sc_gather(ids_flat, neigh_table, features):
    mesh = plsc.VectorSubcoreMesh(core_axis_name="c", subcore_axis_name="s")
    ntp = jnp.zeros((N_NODES, 128), jnp.int32).at[:, :S_NEIGH].set(neigh_table)
    kern = functools.partial(
        pl.kernel, mesh=mesh,
        out_type=jax.ShapeDtypeStruct((ROWS, 2 * D_FEAT), jnp.float32),
        scratch_types=[
            pltpu.VMEM((RPW,), jnp.int32),
            pltpu.VMEM((CH, 128), jnp.int32),
            pltpu.VMEM((CH,), jnp.int32),
            pltpu.VMEM((2, 128), jnp.int32),
            pltpu.VMEM((CH, D_FEAT), jnp.float32),
            pltpu.VMEM((CH * S_NEIGH, D_FEAT), jnp.float32),
            pltpu.VMEM((CH, 2 * D_FEAT), jnp.float32),
            pltpu.SemaphoreType.DMA,
            pltpu.SemaphoreType.DMA,
            pltpu.SemaphoreType.DMA,
        ],
    )(_sc_gather_body)
    return kern(ids_flat, ntp, features)


def _tc_loss_body(g_ref, w1_ref, wasp_ref, out_ref):
    i = pl.program_id(0)

    g = g_ref[...]
    h = jnp.dot(g[:, :D_FEAT], w1_ref[:D_FEAT, :],
                preferred_element_type=jnp.float32)
    h = h + jnp.dot(g[:, D_FEAT:], w1_ref[D_FEAT:, :],
                    preferred_element_type=jnp.float32) * (1.0 / S_NEIGH)
    h = jnp.maximum(h, 0.0)

    locs = []
    for k in range(K_ASP):
        lk = jnp.dot(h, wasp_ref[k], preferred_element_type=jnp.float32)
        locs.append(lk.reshape(BB, M_PAD, D_OUT))

    midx = lax.broadcasted_iota(jnp.int32, (BB, M_PAD), 1)
    valid = (midx >= 1) & (midx <= 1 + NUM_NEGS)

    ls = jnp.zeros((BB, M_PAD), jnp.float32)
    gmax_self = locs[0][:, 0:1, :]
    gmax_all = locs[0]
    for k in range(K_ASP):
        ls = ls + jnp.sum(locs[k][:, 0:1, :] * locs[k], axis=-1)
        if k > 0:
            gmax_self = jnp.maximum(gmax_self, locs[k][:, 0:1, :])
            gmax_all = jnp.maximum(gmax_all, locs[k])
    ls = ls * (1.0 / K_ASP)
    gs = jnp.sum(gmax_self * gmax_all, axis=-1)

    def xent(scores):
        sm = jnp.where(valid, scores, -1e30)
        rmax = jnp.max(sm, axis=1, keepdims=True)
        se = jnp.sum(jnp.where(valid, jnp.exp(scores - rmax), 0.0),
                     axis=1, keepdims=True)
        row = jnp.log(se) + rmax - scores[:, 1:2]
        return jnp.sum(row) * (1.0 / BATCH)

    xent_g = xent(gs)
    xent_l = xent(ls)

    gram = [[None] * K_ASP for _ in range(K_ASP)]
    for k in range(K_ASP):
        for n in range(k, K_ASP):
            p = jnp.sum(locs[k] * locs[n], axis=-1)
            gram[k][n] = p
            gram[n][k] = p
    acc = jnp.zeros((BB, M_PAD), jnp.float32)
    for n in range(K_ASP):
        deno = gram[0][n]
        for k in range(1, K_ASP):
            deno = jnp.maximum(deno, gram[k][n])
        deno = jnp.where(deno == 0.0, 1.0, deno)
        inv = 1.0 / deno
        for k in range(K_ASP):
            tgt = 1.0 if k == n else 0.0
            acc = acc + jnp.abs(gram[k][n] * inv - tgt)
    w = jnp.where(midx == 0, 1.0 / BATCH,
                  jnp.where(valid, 1.0 / (BATCH * (1 + NUM_NEGS)), 0.0))
    constrain = jnp.sum(acc * w)

    contrib = ALPHA * xent_g + BETA * xent_l + GAMMA * constrain

    @pl.when(i == 0)
    def _():
        out_ref[...] = jnp.zeros((1, 1), jnp.float32)
    out_ref[...] = out_ref[...] + jnp.reshape(contrib, (1, 1))


def _tc_loss(G, W1, W_asp, interpret=False):
    return pl.pallas_call(
        _tc_loss_body,
        grid=(GRID,),
        in_specs=[
            pl.BlockSpec((TC_ROWS, 2 * D_FEAT), lambda i: (i, 0)),
            pl.BlockSpec((2 * D_FEAT, D_HIDDEN), lambda i: (0, 0)),
            pl.BlockSpec((K_ASP, D_HIDDEN, D_OUT), lambda i: (0, 0, 0)),
        ],
        out_specs=pl.BlockSpec((1, 1), lambda i: (0, 0)),
        out_shape=jax.ShapeDtypeStruct((1, 1), jnp.float32),
        interpret=interpret,
    )(G, W1, W_asp)


def kernel(edges, negs, neigh_table, features, W1, W_asp):
    pad = jnp.zeros((BATCH, M_PAD - 2 - NUM_NEGS), jnp.int32)
    ids24 = jnp.concatenate(
        [edges[:, 0:1], edges[:, 1:2], negs, pad], axis=1)
    ids_flat = ids24.reshape(ROWS)
    G = _sc_gather(ids_flat, neigh_table, features)
    loss = _tc_loss(G, W1, W_asp)
    return loss[0, 0]


# speedup vs baseline: 2.6358x; 1.1830x over previous
import functools

import jax
import jax.numpy as jnp
from jax import lax
from jax.experimental import pallas as pl
from jax.experimental.pallas import tpu as pltpu
from jax.experimental.pallas import tpu_sc as plsc

N_NODES = 10000
D_FEAT = 256
S_NEIGH = 16
D_HIDDEN = 256
K_ASP = 8
D_OUT = 128
NUM_NEGS = 20
BATCH = 256
ALPHA = 1.0
BETA = 1.0
GAMMA = 0.1

M_PAD = 24
ROWS = BATCH * M_PAD
NUM_CORES = 2
NUM_SUBCORES = 16
NW = NUM_CORES * NUM_SUBCORES
RPW = ROWS // NW
CH = 8
NCH = RPW // CH
LANES = 16

BB = 32
TC_ROWS = BB * M_PAD
GRID = BATCH // BB


def _sc_gather_body(ids_hbm, ntp_hbm, feat_hbm, out_hbm,
                    idx_v,
                    nidc0_v, nidc1_v, fidx0_v, fidx1_v,
                    selfb0_v, selfb1_v, nrows0_v, nrows1_v, outc0_v, outc1_v,
                    sem_n0, sem_n1, sem_f0, sem_f1, sem_o0, sem_o1):
    wid = lax.axis_index("s") * NUM_CORES + lax.axis_index("c")
    base = wid * RPW

    nidc = (nidc0_v, nidc1_v)
    fidx = (fidx0_v, fidx1_v)
    selfb = (selfb0_v, selfb1_v)
    nrows = (nrows0_v, nrows1_v)
    outc = (outc0_v, outc1_v)
    sem_n = (sem_n0, sem_n1)
    sem_f = (sem_f0, sem_f1)
    sem_o = (sem_o0, sem_o1)

    pltpu.sync_copy(ids_hbm.at[pl.ds(base, RPW)], idx_v)

    def issue_nid(c):
        b = c % 2
        return pltpu.async_copy(
            ntp_hbm.at[idx_v.at[pl.ds(c * CH, CH)]], nidc[b], sem_n[b])

    def issue_feat(c):
        b = c % 2
        for j in range(CH):
            fidx[b][0, pl.ds(j * LANES, LANES)] = nidc[b][j, :LANES]
        return (
            pltpu.async_copy(feat_hbm.at[idx_v.at[pl.ds(c * CH, CH)]],
                             selfb[b], sem_f[b]),
            pltpu.async_copy(feat_hbm.at[fidx[b].at[0]], nrows[b], sem_f[b]),
        )

    def reduce(c):
        b = c % 2

        def slot(j, carry2):
            def col(g, carry3):
                sl = pl.ds(g * LANES, LANES)
                acc = nrows[b][j * S_NEIGH, sl]
                for s in range(1, S_NEIGH):
                    acc = acc + nrows[b][j * S_NEIGH + s, sl]
                outc[b][j, pl.ds(D_FEAT + g * LANES, LANES)] = acc
                outc[b][j, sl] = selfb[b][j, sl]
                return carry3
            return lax.fori_loop(0, D_FEAT // LANES, col, carry2)
        lax.fori_loop(0, CH, slot, 0)
        return pltpu.async_copy(
            outc[b], out_hbm.at[pl.ds(base + c * CH, CH), :], sem_o[b])

    hn = {}
    hf = {}
    ho = {}
    hn[0] = issue_nid(0)
    hn[0].wait()
    hf[0] = issue_feat(0)
    hn[1] = issue_nid(1)
    for c in range(NCH):
        for h in hf[c]:
            h.wait()
        if c + 2 < NCH:
            hn[c + 2] = issue_nid(c + 2)
        if c + 1 < NCH:
            hn[c + 1].wait()
            hf[c + 1] = issue_feat(c + 1)
        if c >= 2:
            ho[c - 2].wait()
        ho[c] = reduce(c)
    ho[NCH - 2].wait()
    ho[NCH - 1].wait()


@functools.partial(jax.jit, static_argnums=())
def _sc_gather(ids_flat, neigh_table, features):
    mesh = plsc.VectorSubcoreMesh(core_axis_name="c", subcore_axis_name="s")
    ntp = jnp.zeros((N_NODES, 128), jnp.int32).at[:, :S_NEIGH].set(neigh_table)
    kern = functools.partial(
        pl.kernel, mesh=mesh,
        out_type=jax.ShapeDtypeStruct((ROWS, 2 * D_FEAT), jnp.float32),
        scratch_types=[
            pltpu.VMEM((RPW,), jnp.int32),
            pltpu.VMEM((CH, 128), jnp.int32),
            pltpu.VMEM((CH, 128), jnp.int32),
            pltpu.VMEM((1, 128), jnp.int32),
            pltpu.VMEM((1, 128), jnp.int32),
            pltpu.VMEM((CH, D_FEAT), jnp.float32),
            pltpu.VMEM((CH, D_FEAT), jnp.float32),
            pltpu.VMEM((CH * S_NEIGH, D_FEAT), jnp.float32),
            pltpu.VMEM((CH * S_NEIGH, D_FEAT), jnp.float32),
            pltpu.VMEM((CH, 2 * D_FEAT), jnp.float32),
            pltpu.VMEM((CH, 2 * D_FEAT), jnp.float32),
            pltpu.SemaphoreType.DMA,
            pltpu.SemaphoreType.DMA,
            pltpu.SemaphoreType.DMA,
            pltpu.SemaphoreType.DMA,
            pltpu.SemaphoreType.DMA,
            pltpu.SemaphoreType.DMA,
        ],
    )(_sc_gather_body)
    return kern(ids_flat, ntp, features)


def _tc_loss_body(g_ref, w1_ref, wasp_ref, out_ref):
    i = pl.program_id(0)

    g = g_ref[...]
    h = jnp.dot(g[:, :D_FEAT], w1_ref[:D_FEAT, :],
                preferred_element_type=jnp.float32)
    h = h + jnp.dot(g[:, D_FEAT:], w1_ref[D_FEAT:, :],
                    preferred_element_type=jnp.float32) * (1.0 / S_NEIGH)
    h = jnp.maximum(h, 0.0)

    locs = []
    for k in range(K_ASP):
        lk = jnp.dot(h, wasp_ref[k], preferred_element_type=jnp.float32)
        locs.append(lk.reshape(BB, M_PAD, D_OUT))

    midx = lax.broadcasted_iota(jnp.int32, (BB, M_PAD), 1)
    valid = (midx >= 1) & (midx <= 1 + NUM_NEGS)

    ls = jnp.zeros((BB, M_PAD), jnp.float32)
    gmax_self = locs[0][:, 0:1, :]
    gmax_all = locs[0]
    for k in range(K_ASP):
        ls = ls + jnp.sum(locs[k][:, 0:1, :] * locs[k], axis=-1)
        if k > 0:
            gmax_self = jnp.maximum(gmax_self, locs[k][:, 0:1, :])
            gmax_all = jnp.maximum(gmax_all, locs[k])
    ls = ls * (1.0 / K_ASP)
    gs = jnp.sum(gmax_self * gmax_all, axis=-1)

    def xent(scores):
        sm = jnp.where(valid, scores, -1e30)
        rmax = jnp.max(sm, axis=1, keepdims=True)
        se = jnp.sum(jnp.where(valid, jnp.exp(scores - rmax), 0.0),
                     axis=1, keepdims=True)
        row = jnp.log(se) + rmax - scores[:, 1:2]
        return jnp.sum(row) * (1.0 / BATCH)

    xent_g = xent(gs)
    xent_l = xent(ls)

    gram = [[None] * K_ASP for _ in range(K_ASP)]
    for k in range(K_ASP):
        for n in range(k, K_ASP):
            p = jnp.sum(locs[k] * locs[n], axis=-1)
            gram[k][n] = p
            gram[n][k] = p
    acc = jnp.zeros((BB, M_PAD), jnp.float32)
    for n in range(K_ASP):
        deno = gram[0][n]
        for k in range(1, K_ASP):
            deno = jnp.maximum(deno, gram[k][n])
        deno = jnp.where(deno == 0.0, 1.0, deno)
        inv = 1.0 / deno
        for k in range(K_ASP):
            tgt = 1.0 if k == n else 0.0
            acc = acc + jnp.abs(gram[k][n] * inv - tgt)
    w = jnp.where(midx == 0, 1.0 / BATCH,
                  jnp.where(valid, 1.0 / (BATCH * (1 + NUM_NEGS)), 0.0))
    constrain = jnp.sum(acc * w)

    contrib = ALPHA * xent_g + BETA * xent_l + GAMMA * constrain

    @pl.when(i == 0)
    def _():
        out_ref[...] = jnp.zeros((1, 1), jnp.float32)
    out_ref[...] = out_ref[...] + jnp.reshape(contrib, (1, 1))


def _tc_loss(G, W1, W_asp, interpret=False):
    return pl.pallas_call(
        _tc_loss_body,
        grid=(GRID,),
        in_specs=[
            pl.BlockSpec((TC_ROWS, 2 * D_FEAT), lambda i: (i, 0)),
            pl.BlockSpec((2 * D_FEAT, D_HIDDEN), lambda i: (0, 0)),
            pl.BlockSpec((K_ASP, D_HIDDEN, D_OUT), lambda i: (0, 0, 0)),
        ],
        out_specs=pl.BlockSpec((1, 1), lambda i: (0, 0)),
        out_shape=jax.ShapeDtypeStruct((1, 1), jnp.float32),
        interpret=interpret,
    )(G, W1, W_asp)


def kernel(edges, negs, neigh_table, features, W1, W_asp):
    pad = jnp.zeros((BATCH, M_PAD - 2 - NUM_NEGS), jnp.int32)
    ids24 = jnp.concatenate(
        [edges[:, 0:1], edges[:, 1:2], negs, pad], axis=1)
    ids_flat = ids24.reshape(ROWS)
    G = _sc_gather(ids_flat, neigh_table, features)
    loss = _tc_loss(G, W1, W_asp)
    return loss[0, 0]
